# Initial kernel scaffold; baseline (speedup 1.0000x reference)
import jax
import jax.numpy as jnp
from jax.experimental import pallas as pl


def kernel(x, edge_index, edge_weight, cat_list, Wl, bl, Wr, br, edge_emb):
    raise NotImplementedError("write your pallas kernel here")


# speedup vs baseline: 1.5212x; 1.5212x over previous
import functools

import jax
import jax.numpy as jnp
from jax import lax
from jax.experimental import pallas as pl
from jax.experimental.pallas import tpu as pltpu
from jax.experimental.pallas import tpu_sc as plsc

N = 10000
E = 320000
D = 128

NC = 2
NS = 16
NW = NC * NS
C = 128
GRP = 16
NGRP = 5
EW = C * GRP * NGRP
EPAD = NW * EW
NROW = EPAD // C
A = 10240
RPS = A // NS


def _sc_prop_kernel(src_hbm, dst_hbm, w_hbm, x_hbm, emb_hbm, out_hbm,
                    src_v, dst_v, w_v, buf, emb_buf, prop_sh, sem1, sem2):
    c = lax.axis_index("c")
    s = lax.axis_index("s")
    wid = s * NC + c

    zero = jnp.zeros((16,), jnp.float32)

    def zrow(r, carry):
        for k in range(8):
            buf[r, pl.ds(k * 16, 16)] = zero
        return carry

    lax.fori_loop(0, C, zrow, 0)
    rbase = s * RPS
    for j in range(RPS // C):
        pltpu.sync_copy(buf, prop_sh.at[pl.ds(rbase + j * C, C)])
    plsc.subcore_barrier()

    row0 = wid * (GRP * NGRP)

    def group(grp, carry):
        rb = row0 + grp * GRP
        pltpu.sync_copy(src_hbm.at[pl.ds(rb, GRP)], src_v)
        pltpu.sync_copy(dst_hbm.at[pl.ds(rb, GRP)], dst_v)
        pltpu.sync_copy(w_hbm.at[pl.ds(rb, GRP)], w_v)

        def chunk(g, carry2):
            g1 = pltpu.async_copy(x_hbm.at[src_v.at[g]], buf, sem1)
            g2 = pltpu.async_copy(emb_hbm.at[w_v.at[g]], emb_buf, sem2)
            g1.wait()
            g2.wait()

            def mrow(r, carry3):
                for k in range(8):
                    sl = pl.ds(k * 16, 16)
                    buf[r, sl] = buf[r, sl] * emb_buf[r, sl]
                return carry3

            lax.fori_loop(0, C, mrow, 0)
            pltpu.sync_copy(buf, prop_sh.at[dst_v.at[g]], add=True)
            return carry2

        lax.fori_loop(0, GRP, chunk, 0)
        return carry

    lax.fori_loop(0, NGRP, group, 0)
    plsc.subcore_barrier()

    pltpu.sync_copy(prop_sh.at[pl.ds(rbase, RPS)],
                    out_hbm.at[c, pl.ds(rbase, RPS)])


def _tc_out_kernel(x_ref, p0_ref, p1_ref, wl_ref, wr_ref, b_ref, o_ref):
    acc = lax.dot_general(x_ref[...], wl_ref[...],
                          (((1,), (1,)), ((), ())),
                          preferred_element_type=jnp.float32)
    acc = acc + lax.dot_general(p0_ref[...] + p1_ref[...], wr_ref[...],
                                (((1,), (1,)), ((), ())),
                                preferred_element_type=jnp.float32)
    o_ref[...] = acc + b_ref[...]


def kernel(x, edge_index, edge_weight, cat_list, Wl, bl, Wr, br, edge_emb):
    del cat_list
    pad = EPAD - E
    src = jnp.pad(edge_index[0].astype(jnp.int32), (0, pad)).reshape(NROW, C)
    dst = jnp.pad(edge_index[1].astype(jnp.int32), (0, pad),
                  constant_values=N).reshape(NROW, C)
    w = jnp.pad(edge_weight.astype(jnp.int32), (0, pad)).reshape(NROW, C)
    x = x.astype(jnp.float32)

    sc = functools.partial(
        pl.kernel,
        out_type=jax.ShapeDtypeStruct((2, A, D), jnp.float32),
        mesh=plsc.VectorSubcoreMesh(core_axis_name="c", subcore_axis_name="s"),
        scratch_types=[
            pltpu.VMEM((GRP, C), jnp.int32),
            pltpu.VMEM((GRP, C), jnp.int32),
            pltpu.VMEM((GRP, C), jnp.int32),
            pltpu.VMEM((C, D), jnp.float32),
            pltpu.VMEM((C, D), jnp.float32),
            pltpu.VMEM_SHARED((A, D), jnp.float32),
            pltpu.SemaphoreType.DMA,
            pltpu.SemaphoreType.DMA,
        ],
    )(_sc_prop_kernel)
    prop2 = sc(src, dst, w, x, edge_emb)

    bias = (bl + br).astype(jnp.float32)[None, :]
    BLK = 2000
    nblk = N // BLK
    out = pl.pallas_call(
        _tc_out_kernel,
        grid=(nblk,),
        in_specs=[
            pl.BlockSpec((BLK, D), lambda i: (i, 0)),
            pl.BlockSpec((None, BLK, D), lambda i: (0, i, 0)),
            pl.BlockSpec((None, BLK, D), lambda i: (1, i, 0)),
            pl.BlockSpec((D, D), lambda i: (0, 0)),
            pl.BlockSpec((D, D), lambda i: (0, 0)),
            pl.BlockSpec((1, D), lambda i: (0, 0)),
        ],
        out_specs=pl.BlockSpec((BLK, D), lambda i: (i, 0)),
        out_shape=jax.ShapeDtypeStruct((N, D), jnp.float32),
    )(x, prop2, prop2, Wl, Wr, bias)
    return out


# speedup vs baseline: 4.8387x; 3.1808x over previous
import functools

import jax
import jax.numpy as jnp
from jax import lax
from jax.experimental import pallas as pl
from jax.experimental.pallas import tpu as pltpu
from jax.experimental.pallas import tpu_sc as plsc

N = 10000
E = 320000
D = 128
W = 10

NC = 2
NS = 16
NW = NC * NS
C = 128
CW = 80
GRP = 40
NGRP = CW // GRP
EW = C * CW
EPAD = NW * EW
NROW = EPAD // C
RW = NROW // NW
A = 10240
RPS = A // NS


def _sc_prop_kernel(pk_hbm, xs_hbm, out_hbm,
                    pk_v, comb_v, dst_v, buf_a, buf_b, prop_sh,
                    sga, sgb, ssa, ssb):
    c = lax.axis_index("c")
    s = lax.axis_index("s")
    wid = s * NC + c

    zero = jnp.zeros((16,), jnp.float32)

    def zrow(r, carry):
        for k in range(8):
            buf_a[r, pl.ds(k * 16, 16)] = zero
        return carry

    lax.fori_loop(0, C, zrow, 0)
    rbase = s * RPS
    for j in range(RPS // C):
        pltpu.sync_copy(buf_a, prop_sh.at[pl.ds(rbase + j * C, C)])
    plsc.subcore_barrier()

    row0 = wid * RW

    def gather(g, buf, sem):
        return pltpu.async_copy(xs_hbm.at[comb_v.at[g]], buf, sem)

    def scatter(g, buf, sem):
        return pltpu.async_copy(buf, prop_sh.at[dst_v.at[g]], sem, add=True)

    def wait_gather(buf, sem):
        pltpu.make_async_copy(xs_hbm.at[comb_v.at[0]], buf, sem).wait()

    def wait_scatter(buf, sem):
        pltpu.make_async_copy(buf, prop_sh.at[dst_v.at[0]], sem).wait()

    for grp in range(NGRP):
        pltpu.sync_copy(pk_hbm.at[pl.ds(row0 + grp * GRP, GRP)], pk_v)

        def urow(r, carry):
            for k in range(8):
                sl = pl.ds(k * 16, 16)
                pk = pk_v[r, sl]
                comb_v[r, sl] = lax.bitwise_and(pk, jnp.int32(0x1FFFF))
            for k in range(8):
                sl = pl.ds(k * 16, 16)
                pk = pk_v[r, sl]
                dst_v[r, sl] = lax.shift_right_logical(pk, jnp.int32(17))
            return carry

        lax.fori_loop(0, GRP, urow, 0)

        gather(0, buf_a, sga)

        def pair(p, carry):
            g0 = 2 * p
            wait_gather(buf_a, sga)

            @pl.when(p > 0)
            def _():
                wait_scatter(buf_b, ssb)

            gather(g0 + 1, buf_b, sgb)
            scatter(g0, buf_a, ssa)

            wait_gather(buf_b, sgb)

            @pl.when(p < GRP // 2 - 1)
            def _():
                wait_scatter(buf_a, ssa)
                gather(g0 + 2, buf_a, sga)

            scatter(g0 + 1, buf_b, ssb)
            return carry

        lax.fori_loop(0, GRP // 2, pair, 0)
        wait_scatter(buf_a, ssa)
        wait_scatter(buf_b, ssb)

    plsc.subcore_barrier()

    pltpu.sync_copy(prop_sh.at[pl.ds(rbase, RPS)],
                    out_hbm.at[c, pl.ds(rbase, RPS)])


def _tc_scale_kernel(x_ref, e_ref, o_ref):
    v = pl.program_id(0)
    o_ref[...] = x_ref[...] * e_ref[pl.ds(v, 1), :]


def _tc_out_kernel(x_ref, p0_ref, p1_ref, wl_ref, wr_ref, b_ref, o_ref):
    acc = lax.dot_general(x_ref[...], wl_ref[...],
                          (((1,), (1,)), ((), ())),
                          preferred_element_type=jnp.float32)
    acc = acc + lax.dot_general(p0_ref[...] + p1_ref[...], wr_ref[...],
                                (((1,), (1,)), ((), ())),
                                preferred_element_type=jnp.float32)
    o_ref[...] = acc + b_ref[...]


def kernel(x, edge_index, edge_weight, cat_list, Wl, bl, Wr, br, edge_emb):
    del cat_list
    x = x.astype(jnp.float32)
    pad = EPAD - E
    src = edge_index[0].astype(jnp.int32)
    w = edge_weight.astype(jnp.int32)
    dst = edge_index[1].astype(jnp.int32)
    packed = jnp.pad((w * N + src) | (dst << 17), (0, pad),
                     constant_values=N << 17).reshape(NROW, C)

    BLK = 2000
    nblk = N // BLK
    xs = pl.pallas_call(
        _tc_scale_kernel,
        grid=(W, nblk),
        in_specs=[
            pl.BlockSpec((BLK, D), lambda v, i: (i, 0)),
            pl.BlockSpec((W, D), lambda v, i: (0, 0)),
        ],
        out_specs=pl.BlockSpec((BLK, D), lambda v, i: (v * nblk + i, 0)),
        out_shape=jax.ShapeDtypeStruct((W * N, D), jnp.float32),
    )(x, edge_emb)

    sc = functools.partial(
        pl.kernel,
        out_type=jax.ShapeDtypeStruct((2, A, D), jnp.float32),
        mesh=plsc.VectorSubcoreMesh(core_axis_name="c", subcore_axis_name="s"),
        scratch_types=[
            pltpu.VMEM((GRP, C), jnp.int32),
            pltpu.VMEM((GRP, C), jnp.int32),
            pltpu.VMEM((GRP, C), jnp.int32),
            pltpu.VMEM((C, D), jnp.float32),
            pltpu.VMEM((C, D), jnp.float32),
            pltpu.VMEM_SHARED((A, D), jnp.float32),
            pltpu.SemaphoreType.DMA,
            pltpu.SemaphoreType.DMA,
            pltpu.SemaphoreType.DMA,
            pltpu.SemaphoreType.DMA,
        ],
    )(_sc_prop_kernel)
    prop2 = sc(packed, xs)

    bias = (bl + br).astype(jnp.float32)[None, :]
    out = pl.pallas_call(
        _tc_out_kernel,
        grid=(nblk,),
        in_specs=[
            pl.BlockSpec((BLK, D), lambda i: (i, 0)),
            pl.BlockSpec((None, BLK, D), lambda i: (0, i, 0)),
            pl.BlockSpec((None, BLK, D), lambda i: (1, i, 0)),
            pl.BlockSpec((D, D), lambda i: (0, 0)),
            pl.BlockSpec((D, D), lambda i: (0, 0)),
            pl.BlockSpec((1, D), lambda i: (0, 0)),
        ],
        out_specs=pl.BlockSpec((BLK, D), lambda i: (i, 0)),
        out_shape=jax.ShapeDtypeStruct((N, D), jnp.float32),
    )(x, prop2, prop2, Wl, Wr, bias)
    return out


# speedup vs baseline: 12.4236x; 2.5675x over previous
import functools

import jax
import jax.numpy as jnp
from jax import lax
from jax.experimental import pallas as pl
from jax.experimental.pallas import tpu as pltpu
from jax.experimental.pallas import tpu_sc as plsc

N = 10000
E = 320000
D = 128
W = 10

NC = 2
NS = 16
NW = NC * NS
C = 128
CW = 80
GRP = 40
NGRP = CW // GRP
EW = C * CW
EPAD = NW * EW
NROW = EPAD // C
RW = NROW // NW
A = 10240
RPS = A // NS


def _sc_prop_kernel(pk_hbm, xs_hbm, out_hbm,
                    pk_v, comb_v, dst_v, buf_a, buf_b, prop_sh,
                    sga, sgb, ssa, ssb):
    c = lax.axis_index("c")
    s = lax.axis_index("s")
    wid = s * NC + c

    zero = jnp.zeros((16,), jnp.float32)

    def zrow(r, carry):
        for k in range(8):
            buf_a[r, pl.ds(k * 16, 16)] = zero
        return carry

    lax.fori_loop(0, C, zrow, 0)
    rbase = s * RPS
    for j in range(RPS // C):
        pltpu.sync_copy(buf_a, prop_sh.at[pl.ds(rbase + j * C, C)])
    plsc.subcore_barrier()

    row0 = wid * RW

    def gather(g, buf, sem):
        return pltpu.async_copy(xs_hbm.at[comb_v.at[g]], buf, sem)

    def scatter(g, buf, sem):
        return pltpu.async_copy(buf, prop_sh.at[dst_v.at[g]], sem, add=True)

    def wait_gather(buf, sem):
        pltpu.make_async_copy(xs_hbm.at[comb_v.at[0]], buf, sem).wait()

    def wait_scatter(buf, sem):
        pltpu.make_async_copy(buf, prop_sh.at[dst_v.at[0]], sem).wait()

    for grp in range(NGRP):
        pltpu.sync_copy(pk_hbm.at[pl.ds(row0 + grp * GRP, GRP)], pk_v)

        def urow(r, carry):
            for k in range(8):
                sl = pl.ds(k * 16, 16)
                pk = pk_v[r, sl]
                comb_v[r, sl] = lax.bitwise_and(pk, jnp.int32(0x1FFFF))
            for k in range(8):
                sl = pl.ds(k * 16, 16)
                pk = pk_v[r, sl]
                dst_v[r, sl] = lax.shift_right_logical(pk, jnp.int32(17))
            return carry

        lax.fori_loop(0, GRP, urow, 0)

        gather(0, buf_a, sga)

        def pair(p, carry):
            g0 = 2 * p
            wait_gather(buf_a, sga)

            @pl.when(p > 0)
            def _():
                wait_scatter(buf_b, ssb)

            gather(g0 + 1, buf_b, sgb)
            scatter(g0, buf_a, ssa)

            wait_gather(buf_b, sgb)

            @pl.when(p < GRP // 2 - 1)
            def _():
                wait_scatter(buf_a, ssa)
                gather(g0 + 2, buf_a, sga)

            scatter(g0 + 1, buf_b, ssb)
            return carry

        lax.fori_loop(0, GRP // 2, pair, 0)
        wait_scatter(buf_a, ssa)
        wait_scatter(buf_b, ssb)

    plsc.subcore_barrier()

    pltpu.sync_copy(prop_sh.at[pl.ds(rbase, RPS)],
                    out_hbm.at[c, pl.ds(rbase, RPS)])


def _tc_scale_kernel(x_ref, e_ref, o_ref):
    v = pl.program_id(0)
    o_ref[...] = x_ref[...] * e_ref[pl.ds(v, 1), :]


def _tc_out_kernel(x_ref, p0_ref, p1_ref, wl_ref, wr_ref, b_ref, o_ref):
    acc = lax.dot_general(x_ref[...], wl_ref[...],
                          (((1,), (1,)), ((), ())),
                          preferred_element_type=jnp.float32)
    acc = acc + lax.dot_general(p0_ref[...] + p1_ref[...], wr_ref[...],
                                (((1,), (1,)), ((), ())),
                                preferred_element_type=jnp.float32)
    o_ref[...] = acc + b_ref[...]


def kernel(x, edge_index, edge_weight, cat_list, Wl, bl, Wr, br, edge_emb):
    del cat_list
    x = x.astype(jnp.float32)
    pad = EPAD - E
    src = edge_index[0].astype(jnp.int32)
    w = edge_weight.astype(jnp.int32)
    dst = edge_index[1].astype(jnp.int32)
    pad_idx = jnp.arange(pad, dtype=jnp.int32)
    pad_packed = (pad_idx % N) | ((N + pad_idx % (A - N)) << 17)
    packed = jnp.concatenate(
        [(w * N + src) | (dst << 17), pad_packed]).reshape(NROW, C)

    BLK = 2000
    nblk = N // BLK
    xs = pl.pallas_call(
        _tc_scale_kernel,
        grid=(W, nblk),
        in_specs=[
            pl.BlockSpec((BLK, D), lambda v, i: (i, 0)),
            pl.BlockSpec((W, D), lambda v, i: (0, 0)),
        ],
        out_specs=pl.BlockSpec((BLK, D), lambda v, i: (v * nblk + i, 0)),
        out_shape=jax.ShapeDtypeStruct((W * N, D), jnp.float32),
    )(x, edge_emb)

    sc = functools.partial(
        pl.kernel,
        out_type=jax.ShapeDtypeStruct((2, A, D), jnp.float32),
        mesh=plsc.VectorSubcoreMesh(core_axis_name="c", subcore_axis_name="s"),
        scratch_types=[
            pltpu.VMEM((GRP, C), jnp.int32),
            pltpu.VMEM((GRP, C), jnp.int32),
            pltpu.VMEM((GRP, C), jnp.int32),
            pltpu.VMEM((C, D), jnp.float32),
            pltpu.VMEM((C, D), jnp.float32),
            pltpu.VMEM_SHARED((A, D), jnp.float32),
            pltpu.SemaphoreType.DMA,
            pltpu.SemaphoreType.DMA,
            pltpu.SemaphoreType.DMA,
            pltpu.SemaphoreType.DMA,
        ],
    )(_sc_prop_kernel)
    prop2 = sc(packed, xs)

    bias = (bl + br).astype(jnp.float32)[None, :]
    out = pl.pallas_call(
        _tc_out_kernel,
        grid=(nblk,),
        in_specs=[
            pl.BlockSpec((BLK, D), lambda i: (i, 0)),
            pl.BlockSpec((None, BLK, D), lambda i: (0, i, 0)),
            pl.BlockSpec((None, BLK, D), lambda i: (1, i, 0)),
            pl.BlockSpec((D, D), lambda i: (0, 0)),
            pl.BlockSpec((D, D), lambda i: (0, 0)),
            pl.BlockSpec((1, D), lambda i: (0, 0)),
        ],
        out_specs=pl.BlockSpec((BLK, D), lambda i: (i, 0)),
        out_shape=jax.ShapeDtypeStruct((N, D), jnp.float32),
    )(x, prop2, prop2, Wl, Wr, bias)
    return out


# speedup vs baseline: 13.5776x; 1.0929x over previous
import functools

import jax
import jax.numpy as jnp
from jax import lax
from jax.experimental import pallas as pl
from jax.experimental.pallas import tpu as pltpu
from jax.experimental.pallas import tpu_sc as plsc

N = 10000
E = 320000
D = 128
W = 10

NC = 2
NS = 16
NW = NC * NS
C = 64
CW = 160
GRP = 40
NGRP = CW // GRP
QPG = GRP // 4
EW = C * CW
EPAD = NW * EW
NROW = EPAD // C
RW = NROW // NW
A = 10240
RPS = A // NS


def _sc_prop_kernel(pk_hbm, xs_hbm, out_hbm,
                    pk_v, comb_v, dst_v, b0, b1, b2, b3, prop_sh,
                    sg0, sg1, sg2, sg3, ss0, ss1, ss2, ss3):
    bufs = (b0, b1, b2, b3)
    sgs = (sg0, sg1, sg2, sg3)
    sss = (ss0, ss1, ss2, ss3)
    c = lax.axis_index("c")
    s = lax.axis_index("s")
    wid = s * NC + c

    zero = jnp.zeros((16,), jnp.float32)

    def zrow(r, carry):
        for k in range(8):
            b0[r, pl.ds(k * 16, 16)] = zero
        return carry

    lax.fori_loop(0, C, zrow, 0)
    rbase = s * RPS
    for j in range(RPS // C):
        pltpu.sync_copy(b0, prop_sh.at[pl.ds(rbase + j * C, C)])
    plsc.subcore_barrier()

    row0 = wid * RW

    def gather(g, buf, sem):
        return pltpu.async_copy(xs_hbm.at[comb_v.at[g]], buf, sem)

    def scatter(g, buf, sem):
        return pltpu.async_copy(buf, prop_sh.at[dst_v.at[g]], sem, add=True)

    def wait_gather(buf, sem):
        pltpu.make_async_copy(xs_hbm.at[comb_v.at[0]], buf, sem).wait()

    def wait_scatter(buf, sem):
        pltpu.make_async_copy(buf, prop_sh.at[dst_v.at[0]], sem).wait()

    for grp in range(NGRP):
        pltpu.sync_copy(pk_hbm.at[pl.ds(row0 + grp * GRP, GRP)], pk_v)

        def urow(r, carry):
            for k in range(C // 16):
                sl = pl.ds(k * 16, 16)
                pk = pk_v[r, sl]
                comb_v[r, sl] = lax.bitwise_and(pk, jnp.int32(0x1FFFF))
            for k in range(C // 16):
                sl = pl.ds(k * 16, 16)
                pk = pk_v[r, sl]
                dst_v[r, sl] = lax.shift_right_logical(pk, jnp.int32(17))
            return carry

        lax.fori_loop(0, GRP, urow, 0)

        gather(0, b0, sg0)
        gather(1, b1, sg1)
        gather(2, b2, sg2)

        def quad(q, carry):
            for i in range(4):
                g = 4 * q + i
                j = (i + 3) % 4
                wait_gather(bufs[i], sgs[i])
                scatter(g, bufs[i], sss[i])
                if i == 0:
                    @pl.when(q > 0)
                    def _():
                        wait_scatter(bufs[3], sss[3])

                    gather(g + 3, bufs[3], sgs[3])
                else:
                    @pl.when(q < QPG - 1)
                    def _():
                        wait_scatter(bufs[j], sss[j])
                        gather(g + 3, bufs[j], sgs[j])

            return carry

        lax.fori_loop(0, QPG, quad, 0)
        for i in range(4):
            wait_scatter(bufs[i], sss[i])

    plsc.subcore_barrier()

    pltpu.sync_copy(prop_sh.at[pl.ds(rbase, RPS)],
                    out_hbm.at[c, pl.ds(rbase, RPS)])


def _tc_scale_kernel(x_ref, e_ref, o_ref):
    v = pl.program_id(0)
    o_ref[...] = x_ref[...] * e_ref[pl.ds(v, 1), :]


def _tc_out_kernel(x_ref, p0_ref, p1_ref, wl_ref, wr_ref, b_ref, o_ref):
    acc = lax.dot_general(x_ref[...], wl_ref[...],
                          (((1,), (1,)), ((), ())),
                          preferred_element_type=jnp.float32)
    acc = acc + lax.dot_general(p0_ref[...] + p1_ref[...], wr_ref[...],
                                (((1,), (1,)), ((), ())),
                                preferred_element_type=jnp.float32)
    o_ref[...] = acc + b_ref[...]


def kernel(x, edge_index, edge_weight, cat_list, Wl, bl, Wr, br, edge_emb):
    del cat_list
    x = x.astype(jnp.float32)
    pad = EPAD - E
    src = edge_index[0].astype(jnp.int32)
    w = edge_weight.astype(jnp.int32)
    dst = edge_index[1].astype(jnp.int32)
    pad_idx = jnp.arange(pad, dtype=jnp.int32)
    pad_packed = (pad_idx % N) | ((N + pad_idx % (A - N)) << 17)
    packed = jnp.concatenate(
        [(w * N + src) | (dst << 17), pad_packed]).reshape(NROW, C)

    BLK = 2000
    nblk = N // BLK
    xs = pl.pallas_call(
        _tc_scale_kernel,
        grid=(W, nblk),
        in_specs=[
            pl.BlockSpec((BLK, D), lambda v, i: (i, 0)),
            pl.BlockSpec((W, D), lambda v, i: (0, 0)),
        ],
        out_specs=pl.BlockSpec((BLK, D), lambda v, i: (v * nblk + i, 0)),
        out_shape=jax.ShapeDtypeStruct((W * N, D), jnp.float32),
    )(x, edge_emb)

    sc = functools.partial(
        pl.kernel,
        out_type=jax.ShapeDtypeStruct((2, A, D), jnp.float32),
        mesh=plsc.VectorSubcoreMesh(core_axis_name="c", subcore_axis_name="s"),
        scratch_types=[
            pltpu.VMEM((GRP, C), jnp.int32),
            pltpu.VMEM((GRP, C), jnp.int32),
            pltpu.VMEM((GRP, C), jnp.int32),
            pltpu.VMEM((C, D), jnp.float32),
            pltpu.VMEM((C, D), jnp.float32),
            pltpu.VMEM((C, D), jnp.float32),
            pltpu.VMEM((C, D), jnp.float32),
            pltpu.VMEM_SHARED((A, D), jnp.float32),
            pltpu.SemaphoreType.DMA,
            pltpu.SemaphoreType.DMA,
            pltpu.SemaphoreType.DMA,
            pltpu.SemaphoreType.DMA,
            pltpu.SemaphoreType.DMA,
            pltpu.SemaphoreType.DMA,
            pltpu.SemaphoreType.DMA,
            pltpu.SemaphoreType.DMA,
        ],
    )(_sc_prop_kernel)
    prop2 = sc(packed, xs)

    bias = (bl + br).astype(jnp.float32)[None, :]
    out = pl.pallas_call(
        _tc_out_kernel,
        grid=(nblk,),
        in_specs=[
            pl.BlockSpec((BLK, D), lambda i: (i, 0)),
            pl.BlockSpec((None, BLK, D), lambda i: (0, i, 0)),
            pl.BlockSpec((None, BLK, D), lambda i: (1, i, 0)),
            pl.BlockSpec((D, D), lambda i: (0, 0)),
            pl.BlockSpec((D, D), lambda i: (0, 0)),
            pl.BlockSpec((1, D), lambda i: (0, 0)),
        ],
        out_specs=pl.BlockSpec((BLK, D), lambda i: (i, 0)),
        out_shape=jax.ShapeDtypeStruct((N, D), jnp.float32),
    )(x, prop2, prop2, Wl, Wr, bias)
    return out


# speedup vs baseline: 13.6061x; 1.0021x over previous
import functools

import jax
import jax.numpy as jnp
from jax import lax
from jax.experimental import pallas as pl
from jax.experimental.pallas import tpu as pltpu
from jax.experimental.pallas import tpu_sc as plsc

N = 10000
E = 320000
D = 128
W = 10

NC = 2
NS = 16
NW = NC * NS
C = 64
CW = 160
GRP = 40
NGRP = CW // GRP
QPG = GRP // 4
EW = C * CW
EPAD = NW * EW
NROW = EPAD // C
RW = NROW // NW
A = 10240
RPS = A // NS


def _sc_prop_kernel(pk_hbm, xs_hbm, out_hbm,
                    pk_v, comb_v, dst_v, b0, b1, b2, b3, prop_sh,
                    sg0, sg1, sg2, sg3, ss0, ss1, ss2, ss3):
    bufs = (b0, b1, b2, b3)
    sgs = (sg0, sg1, sg2, sg3)
    sss = (ss0, ss1, ss2, ss3)
    c = lax.axis_index("c")
    s = lax.axis_index("s")
    wid = s * NC + c

    zero = jnp.zeros((16,), jnp.float32)

    def zrow(r, carry):
        for k in range(8):
            b0[r, pl.ds(k * 16, 16)] = zero
        return carry

    lax.fori_loop(0, C, zrow, 0)
    rbase = s * RPS
    for j in range(RPS // C):
        pltpu.async_copy(b0, prop_sh.at[pl.ds(rbase + j * C, C)], sg0)
    for j in range(RPS // C):
        pltpu.make_async_copy(b0, prop_sh.at[pl.ds(rbase, C)], sg0).wait()
    plsc.subcore_barrier()

    row0 = wid * RW

    def gather(g, buf, sem):
        return pltpu.async_copy(xs_hbm.at[comb_v.at[g]], buf, sem)

    def scatter(g, buf, sem):
        return pltpu.async_copy(buf, prop_sh.at[dst_v.at[g]], sem, add=True)

    def wait_gather(buf, sem):
        pltpu.make_async_copy(xs_hbm.at[comb_v.at[0]], buf, sem).wait()

    def wait_scatter(buf, sem):
        pltpu.make_async_copy(buf, prop_sh.at[dst_v.at[0]], sem).wait()

    for grp in range(NGRP):
        pltpu.sync_copy(pk_hbm.at[pl.ds(row0 + grp * GRP, GRP)], pk_v)

        def urow(r, carry):
            for k in range(C // 16):
                sl = pl.ds(k * 16, 16)
                pk = pk_v[r, sl]
                comb_v[r, sl] = lax.bitwise_and(pk, jnp.int32(0x1FFFF))
            for k in range(C // 16):
                sl = pl.ds(k * 16, 16)
                pk = pk_v[r, sl]
                dst_v[r, sl] = lax.shift_right_logical(pk, jnp.int32(17))
            return carry

        lax.fori_loop(0, GRP, urow, 0)

        gather(0, b0, sg0)
        gather(1, b1, sg1)
        gather(2, b2, sg2)

        def quad(q, carry):
            for i in range(4):
                g = 4 * q + i
                j = (i + 3) % 4
                wait_gather(bufs[i], sgs[i])
                scatter(g, bufs[i], sss[i])
                if i == 0:
                    @pl.when(q > 0)
                    def _():
                        wait_scatter(bufs[3], sss[3])

                    gather(g + 3, bufs[3], sgs[3])
                else:
                    @pl.when(q < QPG - 1)
                    def _():
                        wait_scatter(bufs[j], sss[j])
                        gather(g + 3, bufs[j], sgs[j])

            return carry

        lax.fori_loop(0, QPG, quad, 0)
        for i in range(4):
            wait_scatter(bufs[i], sss[i])

    plsc.subcore_barrier()

    pltpu.sync_copy(prop_sh.at[pl.ds(rbase, RPS)],
                    out_hbm.at[c, pl.ds(rbase, RPS)])


def _tc_scale_kernel(x_ref, e_ref, o_ref):
    v = pl.program_id(0)
    o_ref[...] = x_ref[...] * e_ref[pl.ds(v, 1), :]


def _tc_lin_kernel(x_ref, wl_ref, b_ref, o_ref):
    o_ref[...] = lax.dot_general(x_ref[...], wl_ref[...],
                                 (((1,), (1,)), ((), ())),
                                 preferred_element_type=jnp.float32) + b_ref[...]


def _tc_out_kernel(y_ref, p0_ref, p1_ref, wr_ref, o_ref):
    o_ref[...] = y_ref[...] + lax.dot_general(
        p0_ref[...] + p1_ref[...], wr_ref[...],
        (((1,), (1,)), ((), ())), preferred_element_type=jnp.float32)


def kernel(x, edge_index, edge_weight, cat_list, Wl, bl, Wr, br, edge_emb):
    del cat_list
    x = x.astype(jnp.float32)
    pad = EPAD - E
    src = edge_index[0].astype(jnp.int32)
    w = edge_weight.astype(jnp.int32)
    dst = edge_index[1].astype(jnp.int32)
    pad_idx = jnp.arange(pad, dtype=jnp.int32)
    pad_packed = (pad_idx % N) | ((N + pad_idx % (A - N)) << 17)
    packed = jnp.concatenate(
        [(w * N + src) | (dst << 17), pad_packed]).reshape(NROW, C)

    BLK = 2000
    nblk = N // BLK
    xs = pl.pallas_call(
        _tc_scale_kernel,
        grid=(W, nblk),
        in_specs=[
            pl.BlockSpec((BLK, D), lambda v, i: (i, 0)),
            pl.BlockSpec((W, D), lambda v, i: (0, 0)),
        ],
        out_specs=pl.BlockSpec((BLK, D), lambda v, i: (v * nblk + i, 0)),
        out_shape=jax.ShapeDtypeStruct((W * N, D), jnp.float32),
    )(x, edge_emb)

    sc = functools.partial(
        pl.kernel,
        out_type=jax.ShapeDtypeStruct((2, A, D), jnp.float32),
        mesh=plsc.VectorSubcoreMesh(core_axis_name="c", subcore_axis_name="s"),
        scratch_types=[
            pltpu.VMEM((GRP, C), jnp.int32),
            pltpu.VMEM((GRP, C), jnp.int32),
            pltpu.VMEM((GRP, C), jnp.int32),
            pltpu.VMEM((C, D), jnp.float32),
            pltpu.VMEM((C, D), jnp.float32),
            pltpu.VMEM((C, D), jnp.float32),
            pltpu.VMEM((C, D), jnp.float32),
            pltpu.VMEM_SHARED((A, D), jnp.float32),
            pltpu.SemaphoreType.DMA,
            pltpu.SemaphoreType.DMA,
            pltpu.SemaphoreType.DMA,
            pltpu.SemaphoreType.DMA,
            pltpu.SemaphoreType.DMA,
            pltpu.SemaphoreType.DMA,
            pltpu.SemaphoreType.DMA,
            pltpu.SemaphoreType.DMA,
        ],
    )(_sc_prop_kernel)
    prop2 = sc(packed, xs)

    bias = (bl + br).astype(jnp.float32)[None, :]
    y = pl.pallas_call(
        _tc_lin_kernel,
        grid=(nblk,),
        in_specs=[
            pl.BlockSpec((BLK, D), lambda i: (i, 0)),
            pl.BlockSpec((D, D), lambda i: (0, 0)),
            pl.BlockSpec((1, D), lambda i: (0, 0)),
        ],
        out_specs=pl.BlockSpec((BLK, D), lambda i: (i, 0)),
        out_shape=jax.ShapeDtypeStruct((N, D), jnp.float32),
    )(x, Wl, bias)

    out = pl.pallas_call(
        _tc_out_kernel,
        grid=(nblk,),
        in_specs=[
            pl.BlockSpec((BLK, D), lambda i: (i, 0)),
            pl.BlockSpec((None, BLK, D), lambda i: (0, i, 0)),
            pl.BlockSpec((None, BLK, D), lambda i: (1, i, 0)),
            pl.BlockSpec((D, D), lambda i: (0, 0)),
        ],
        out_specs=pl.BlockSpec((BLK, D), lambda i: (i, 0)),
        out_shape=jax.ShapeDtypeStruct((N, D), jnp.float32),
    )(y, prop2, prop2, Wr)
    return out


# speedup vs baseline: 14.2583x; 1.0479x over previous
import functools

import jax
import jax.numpy as jnp
from jax import lax
from jax.experimental import pallas as pl
from jax.experimental.pallas import tpu as pltpu
from jax.experimental.pallas import tpu_sc as plsc

N = 10000
E = 320000
D = 128
W = 10

NC = 2
NS = 16
NW = NC * NS
C = 64
CW = 160
GRP = 40
NGRP = CW // GRP
QPG = GRP // 4
EW = C * CW
EPAD = NW * EW
NROW = EPAD // C
RW = NROW // NW
A = 10240
RPS = A // NS


def _sc_prop_kernel(pk_hbm, xs_hbm, out_hbm,
                    pk_v, comb_v, dst_v, b0, b1, b2, b3, prop_sh,
                    sg0, sg1, sg2, sg3, ss0, ss1, ss2, ss3):
    bufs = (b0, b1, b2, b3)
    sgs = (sg0, sg1, sg2, sg3)
    sss = (ss0, ss1, ss2, ss3)
    c = lax.axis_index("c")
    s = lax.axis_index("s")
    wid = s * NC + c

    zero = jnp.zeros((16,), jnp.float32)

    def zrow(r, carry):
        for k in range(8):
            b0[r, pl.ds(k * 16, 16)] = zero
        return carry

    lax.fori_loop(0, C, zrow, 0)
    rbase = s * RPS
    for j in range(RPS // C):
        pltpu.async_copy(b0, prop_sh.at[pl.ds(rbase + j * C, C)], sg0)
    for j in range(RPS // C):
        pltpu.make_async_copy(b0, prop_sh.at[pl.ds(rbase, C)], sg0).wait()
    plsc.subcore_barrier()

    row0 = wid * RW

    def gather(g, buf, sem):
        return pltpu.async_copy(xs_hbm.at[comb_v.at[g]], buf, sem)

    def scatter(g, buf, sem):
        return pltpu.async_copy(buf, prop_sh.at[dst_v.at[g]], sem, add=True)

    def wait_gather(buf, sem):
        pltpu.make_async_copy(xs_hbm.at[comb_v.at[0]], buf, sem).wait()

    def wait_scatter(buf, sem):
        pltpu.make_async_copy(buf, prop_sh.at[dst_v.at[0]], sem).wait()

    for grp in range(NGRP):
        pltpu.sync_copy(pk_hbm.at[pl.ds(row0 + grp * GRP, GRP)], pk_v)

        def urow(r, carry):
            for k in range(C // 16):
                sl = pl.ds(k * 16, 16)
                pk = pk_v[r, sl]
                comb_v[r, sl] = lax.bitwise_and(pk, jnp.int32(0x1FFFF))
            for k in range(C // 16):
                sl = pl.ds(k * 16, 16)
                pk = pk_v[r, sl]
                dst_v[r, sl] = lax.shift_right_logical(pk, jnp.int32(17))
            return carry

        lax.fori_loop(0, GRP, urow, 0)

        gather(0, b0, sg0)
        gather(1, b1, sg1)
        gather(2, b2, sg2)

        def quad(q, carry):
            for i in range(4):
                g = 4 * q + i
                j = (i + 3) % 4
                wait_gather(bufs[i], sgs[i])
                scatter(g, bufs[i], sss[i])
                if i == 0:
                    @pl.when(q > 0)
                    def _():
                        wait_scatter(bufs[3], sss[3])

                    gather(g + 3, bufs[3], sgs[3])
                else:
                    @pl.when(q < QPG - 1)
                    def _():
                        wait_scatter(bufs[j], sss[j])
                        gather(g + 3, bufs[j], sgs[j])

            return carry

        lax.fori_loop(0, QPG, quad, 0)
        for i in range(4):
            wait_scatter(bufs[i], sss[i])

    plsc.subcore_barrier()

    pltpu.sync_copy(prop_sh.at[pl.ds(rbase, RPS)],
                    out_hbm.at[c, pl.ds(rbase, RPS)])


def _tc_scale_kernel(x_ref, e_ref, wl_ref, b_ref, o_ref, y_ref):
    v = pl.program_id(1)
    o_ref[...] = x_ref[...] * e_ref[pl.ds(v, 1), :]
    y_ref[...] = lax.dot_general(x_ref[...], wl_ref[...],
                                 (((1,), (1,)), ((), ())),
                                 preferred_element_type=jnp.float32) + b_ref[...]


def _tc_out_kernel(y_ref, p0_ref, p1_ref, wr_ref, o_ref):
    o_ref[...] = y_ref[...] + lax.dot_general(
        p0_ref[...] + p1_ref[...], wr_ref[...],
        (((1,), (1,)), ((), ())), preferred_element_type=jnp.float32)


def kernel(x, edge_index, edge_weight, cat_list, Wl, bl, Wr, br, edge_emb):
    del cat_list
    x = x.astype(jnp.float32)
    pad = EPAD - E
    src = edge_index[0].astype(jnp.int32)
    w = edge_weight.astype(jnp.int32)
    dst = edge_index[1].astype(jnp.int32)
    pad_idx = jnp.arange(pad, dtype=jnp.int32)
    pad_packed = (pad_idx % N) | ((N + pad_idx % (A - N)) << 17)
    packed = jnp.concatenate(
        [(w * N + src) | (dst << 17), pad_packed]).reshape(NROW, C)

    bias = (bl + br).astype(jnp.float32)[None, :]
    BLK = 2000
    nblk = N // BLK
    xs, y = pl.pallas_call(
        _tc_scale_kernel,
        grid=(nblk, W),
        in_specs=[
            pl.BlockSpec((BLK, D), lambda i, v: (i, 0)),
            pl.BlockSpec((W, D), lambda i, v: (0, 0)),
            pl.BlockSpec((D, D), lambda i, v: (0, 0)),
            pl.BlockSpec((1, D), lambda i, v: (0, 0)),
        ],
        out_specs=[
            pl.BlockSpec((BLK, D), lambda i, v: (v * nblk + i, 0)),
            pl.BlockSpec((BLK, D), lambda i, v: (i, 0)),
        ],
        out_shape=[
            jax.ShapeDtypeStruct((W * N, D), jnp.float32),
            jax.ShapeDtypeStruct((N, D), jnp.float32),
        ],
    )(x, edge_emb, Wl, bias)

    sc = functools.partial(
        pl.kernel,
        out_type=jax.ShapeDtypeStruct((2, A, D), jnp.float32),
        mesh=plsc.VectorSubcoreMesh(core_axis_name="c", subcore_axis_name="s"),
        scratch_types=[
            pltpu.VMEM((GRP, C), jnp.int32),
            pltpu.VMEM((GRP, C), jnp.int32),
            pltpu.VMEM((GRP, C), jnp.int32),
            pltpu.VMEM((C, D), jnp.float32),
            pltpu.VMEM((C, D), jnp.float32),
            pltpu.VMEM((C, D), jnp.float32),
            pltpu.VMEM((C, D), jnp.float32),
            pltpu.VMEM_SHARED((A, D), jnp.float32),
            pltpu.SemaphoreType.DMA,
            pltpu.SemaphoreType.DMA,
            pltpu.SemaphoreType.DMA,
            pltpu.SemaphoreType.DMA,
            pltpu.SemaphoreType.DMA,
            pltpu.SemaphoreType.DMA,
            pltpu.SemaphoreType.DMA,
            pltpu.SemaphoreType.DMA,
        ],
    )(_sc_prop_kernel)
    prop2 = sc(packed, xs)

    out = pl.pallas_call(
        _tc_out_kernel,
        grid=(nblk,),
        in_specs=[
            pl.BlockSpec((BLK, D), lambda i: (i, 0)),
            pl.BlockSpec((None, BLK, D), lambda i: (0, i, 0)),
            pl.BlockSpec((None, BLK, D), lambda i: (1, i, 0)),
            pl.BlockSpec((D, D), lambda i: (0, 0)),
        ],
        out_specs=pl.BlockSpec((BLK, D), lambda i: (i, 0)),
        out_shape=jax.ShapeDtypeStruct((N, D), jnp.float32),
    )(y, prop2, prop2, Wr)
    return out


# speedup vs baseline: 14.3992x; 1.0099x over previous
import functools

import jax
import jax.numpy as jnp
from jax import lax
from jax.experimental import pallas as pl
from jax.experimental.pallas import tpu as pltpu
from jax.experimental.pallas import tpu_sc as plsc

N = 10000
E = 320000
D = 128
W = 10

NC = 2
NS = 16
NW = NC * NS
C = 64
CW = 160
GRP = 40
NGRP = CW // GRP
QPG = GRP // 4
EW = C * CW
EPAD = NW * EW
NROW = EPAD // C
RW = NROW // NW
A = 10240
RPS = A // NS


def _sc_prop_kernel(pk_hbm, xs_hbm, out_hbm,
                    pk_v, comb_v, dst_v, b0, b1, b2, b3, prop_sh,
                    sg0, sg1, sg2, sg3, ss0, ss1, ss2, ss3):
    bufs = (b0, b1, b2, b3)
    sgs = (sg0, sg1, sg2, sg3)
    sss = (ss0, ss1, ss2, ss3)
    c = lax.axis_index("c")
    s = lax.axis_index("s")
    wid = s * NC + c

    zero = jnp.zeros((16,), jnp.float32)

    def zrow(r, carry):
        for k in range(8):
            b0[r, pl.ds(k * 16, 16)] = zero
        return carry

    lax.fori_loop(0, C, zrow, 0)
    rbase = s * RPS
    for j in range(RPS // C):
        pltpu.async_copy(b0, prop_sh.at[pl.ds(rbase + j * C, C)], ss0)

    row0 = wid * RW

    def gather(g, buf, sem):
        return pltpu.async_copy(xs_hbm.at[comb_v.at[g]], buf, sem)

    def scatter(g, buf, sem):
        return pltpu.async_copy(buf, prop_sh.at[dst_v.at[g]], sem, add=True)

    def wait_gather(buf, sem):
        pltpu.make_async_copy(xs_hbm.at[comb_v.at[0]], buf, sem).wait()

    def wait_scatter(buf, sem):
        pltpu.make_async_copy(buf, prop_sh.at[dst_v.at[0]], sem).wait()

    for grp in range(NGRP):
        pltpu.sync_copy(pk_hbm.at[pl.ds(row0 + grp * GRP, GRP)], pk_v)

        def urow(r, carry):
            for k in range(C // 16):
                sl = pl.ds(k * 16, 16)
                pk = pk_v[r, sl]
                comb_v[r, sl] = lax.bitwise_and(pk, jnp.int32(0x1FFFF))
            for k in range(C // 16):
                sl = pl.ds(k * 16, 16)
                pk = pk_v[r, sl]
                dst_v[r, sl] = lax.shift_right_logical(pk, jnp.int32(17))
            return carry

        lax.fori_loop(0, GRP, urow, 0)

        if grp == 0:
            gather(1, b1, sg1)
            gather(2, b2, sg2)
            for j in range(RPS // C):
                pltpu.make_async_copy(b0, prop_sh.at[pl.ds(rbase, C)],
                                      ss0).wait()
            plsc.subcore_barrier()
            gather(0, b0, sg0)
        else:
            gather(0, b0, sg0)
            gather(1, b1, sg1)
            gather(2, b2, sg2)

        def quad(q, carry):
            for i in range(4):
                g = 4 * q + i
                j = (i + 3) % 4
                wait_gather(bufs[i], sgs[i])
                scatter(g, bufs[i], sss[i])
                if i == 0:
                    @pl.when(q > 0)
                    def _():
                        wait_scatter(bufs[3], sss[3])

                    gather(g + 3, bufs[3], sgs[3])
                else:
                    @pl.when(q < QPG - 1)
                    def _():
                        wait_scatter(bufs[j], sss[j])
                        gather(g + 3, bufs[j], sgs[j])

            return carry

        lax.fori_loop(0, QPG, quad, 0)
        for i in range(4):
            wait_scatter(bufs[i], sss[i])

    plsc.subcore_barrier()

    pltpu.sync_copy(prop_sh.at[pl.ds(rbase, RPS)],
                    out_hbm.at[c, pl.ds(rbase, RPS)])


def _tc_scale_kernel(x_ref, e_ref, wl_ref, b_ref, o_ref, y_ref):
    v = pl.program_id(1)
    o_ref[...] = x_ref[...] * e_ref[pl.ds(v, 1), :]
    y_ref[...] = lax.dot_general(x_ref[...], wl_ref[...],
                                 (((1,), (1,)), ((), ())),
                                 preferred_element_type=jnp.float32) + b_ref[...]


def _tc_out_kernel(y_ref, p0_ref, p1_ref, wr_ref, o_ref):
    o_ref[...] = y_ref[...] + lax.dot_general(
        p0_ref[...] + p1_ref[...], wr_ref[...],
        (((1,), (1,)), ((), ())), preferred_element_type=jnp.float32)


def kernel(x, edge_index, edge_weight, cat_list, Wl, bl, Wr, br, edge_emb):
    del cat_list
    x = x.astype(jnp.float32)
    pad = EPAD - E
    src = edge_index[0].astype(jnp.int32)
    w = edge_weight.astype(jnp.int32)
    dst = edge_index[1].astype(jnp.int32)
    pad_idx = jnp.arange(pad, dtype=jnp.int32)
    pad_packed = (pad_idx % N) | ((N + pad_idx % (A - N)) << 17)
    packed = jnp.concatenate(
        [(w * N + src) | (dst << 17), pad_packed]).reshape(NROW, C)

    bias = (bl + br).astype(jnp.float32)[None, :]
    BLK = 2000
    nblk = N // BLK
    xs, y = pl.pallas_call(
        _tc_scale_kernel,
        grid=(nblk, W),
        in_specs=[
            pl.BlockSpec((BLK, D), lambda i, v: (i, 0)),
            pl.BlockSpec((W, D), lambda i, v: (0, 0)),
            pl.BlockSpec((D, D), lambda i, v: (0, 0)),
            pl.BlockSpec((1, D), lambda i, v: (0, 0)),
        ],
        out_specs=[
            pl.BlockSpec((BLK, D), lambda i, v: (v * nblk + i, 0)),
            pl.BlockSpec((BLK, D), lambda i, v: (i, 0)),
        ],
        out_shape=[
            jax.ShapeDtypeStruct((W * N, D), jnp.float32),
            jax.ShapeDtypeStruct((N, D), jnp.float32),
        ],
    )(x, edge_emb, Wl, bias)

    sc = functools.partial(
        pl.kernel,
        out_type=jax.ShapeDtypeStruct((2, A, D), jnp.float32),
        mesh=plsc.VectorSubcoreMesh(core_axis_name="c", subcore_axis_name="s"),
        scratch_types=[
            pltpu.VMEM((GRP, C), jnp.int32),
            pltpu.VMEM((GRP, C), jnp.int32),
            pltpu.VMEM((GRP, C), jnp.int32),
            pltpu.VMEM((C, D), jnp.float32),
            pltpu.VMEM((C, D), jnp.float32),
            pltpu.VMEM((C, D), jnp.float32),
            pltpu.VMEM((C, D), jnp.float32),
            pltpu.VMEM_SHARED((A, D), jnp.float32),
            pltpu.SemaphoreType.DMA,
            pltpu.SemaphoreType.DMA,
            pltpu.SemaphoreType.DMA,
            pltpu.SemaphoreType.DMA,
            pltpu.SemaphoreType.DMA,
            pltpu.SemaphoreType.DMA,
            pltpu.SemaphoreType.DMA,
            pltpu.SemaphoreType.DMA,
        ],
    )(_sc_prop_kernel)
    prop2 = sc(packed, xs)

    out = pl.pallas_call(
        _tc_out_kernel,
        grid=(nblk,),
        in_specs=[
            pl.BlockSpec((BLK, D), lambda i: (i, 0)),
            pl.BlockSpec((None, BLK, D), lambda i: (0, i, 0)),
            pl.BlockSpec((None, BLK, D), lambda i: (1, i, 0)),
            pl.BlockSpec((D, D), lambda i: (0, 0)),
        ],
        out_specs=pl.BlockSpec((BLK, D), lambda i: (i, 0)),
        out_shape=jax.ShapeDtypeStruct((N, D), jnp.float32),
    )(y, prop2, prop2, Wr)
    return out


# speedup vs baseline: 14.4630x; 1.0044x over previous
import functools

import jax
import jax.numpy as jnp
from jax import lax
from jax.experimental import pallas as pl
from jax.experimental.pallas import tpu as pltpu
from jax.experimental.pallas import tpu_sc as plsc

N = 10000
E = 320000
D = 128
W = 10

NC = 2
NS = 16
NW = NC * NS
C = 64
CW = 160
GRP = 40
NGRP = CW // GRP
QPG = GRP // 4
EW = C * CW
EPAD = NW * EW
NROW = EPAD // C
RW = NROW // NW
A = 10240
RPS = A // NS


def _sc_prop_kernel(pk_hbm, xs_hbm, out_hbm,
                    pk_v, comb_v, dst_v, b0, b1, b2, b3, prop_sh,
                    sg0, sg1, sg2, sg3, ss0, ss1, ss2, ss3, spk):
    bufs = (b0, b1, b2, b3)
    sgs = (sg0, sg1, sg2, sg3)
    sss = (ss0, ss1, ss2, ss3)
    c = lax.axis_index("c")
    s = lax.axis_index("s")
    wid = s * NC + c

    zero = jnp.zeros((16,), jnp.float32)

    def zrow(r, carry):
        for k in range(8):
            b0[r, pl.ds(k * 16, 16)] = zero
        return carry

    lax.fori_loop(0, C, zrow, 0)
    rbase = s * RPS
    for j in range(RPS // C):
        pltpu.async_copy(b0, prop_sh.at[pl.ds(rbase + j * C, C)], ss0)

    row0 = wid * RW

    def gather(g, buf, sem):
        return pltpu.async_copy(xs_hbm.at[comb_v.at[g]], buf, sem)

    def scatter(g, buf, sem):
        return pltpu.async_copy(buf, prop_sh.at[dst_v.at[g]], sem, add=True)

    def wait_gather(buf, sem):
        pltpu.make_async_copy(xs_hbm.at[comb_v.at[0]], buf, sem).wait()

    def wait_scatter(buf, sem):
        pltpu.make_async_copy(buf, prop_sh.at[dst_v.at[0]], sem).wait()

    for grp in range(NGRP):
        if grp == 0:
            pltpu.async_copy(pk_hbm.at[pl.ds(row0, GRP)], pk_v, spk)
        pltpu.make_async_copy(pk_hbm.at[pl.ds(row0, GRP)], pk_v, spk).wait()

        def urow(r, carry):
            for k in range(C // 16):
                sl = pl.ds(k * 16, 16)
                pk = pk_v[r, sl]
                comb_v[r, sl] = lax.bitwise_and(pk, jnp.int32(0x1FFFF))
            for k in range(C // 16):
                sl = pl.ds(k * 16, 16)
                pk = pk_v[r, sl]
                dst_v[r, sl] = lax.shift_right_logical(pk, jnp.int32(17))
            return carry

        lax.fori_loop(0, GRP, urow, 0)

        if grp == 0:
            gather(1, b1, sg1)
            gather(2, b2, sg2)
            for j in range(RPS // C):
                pltpu.make_async_copy(b0, prop_sh.at[pl.ds(rbase, C)],
                                      ss0).wait()
            plsc.subcore_barrier()
            gather(0, b0, sg0)
        else:
            gather(0, b0, sg0)
            gather(1, b1, sg1)
            gather(2, b2, sg2)

        def quad(q, carry):
            for i in range(4):
                g = 4 * q + i
                j = (i + 3) % 4
                wait_gather(bufs[i], sgs[i])
                scatter(g, bufs[i], sss[i])
                if i == 0:
                    @pl.when(q > 0)
                    def _():
                        wait_scatter(bufs[3], sss[3])

                    gather(g + 3, bufs[3], sgs[3])
                else:
                    @pl.when(q < QPG - 1)
                    def _():
                        wait_scatter(bufs[j], sss[j])
                        gather(g + 3, bufs[j], sgs[j])

            return carry

        lax.fori_loop(0, QPG, quad, 0)
        if grp < NGRP - 1:
            pltpu.async_copy(pk_hbm.at[pl.ds(row0 + (grp + 1) * GRP, GRP)],
                             pk_v, spk)
        for i in range(4):
            wait_scatter(bufs[i], sss[i])

    plsc.subcore_barrier()

    pltpu.sync_copy(prop_sh.at[pl.ds(rbase, RPS)],
                    out_hbm.at[c, pl.ds(rbase, RPS)])


def _tc_scale_kernel(x_ref, e_ref, wl_ref, b_ref, o_ref, y_ref):
    v = pl.program_id(1)
    o_ref[...] = x_ref[...] * e_ref[pl.ds(v, 1), :]
    y_ref[...] = lax.dot_general(x_ref[...], wl_ref[...],
                                 (((1,), (1,)), ((), ())),
                                 preferred_element_type=jnp.float32) + b_ref[...]


def _tc_out_kernel(y_ref, p0_ref, p1_ref, wr_ref, o_ref):
    o_ref[...] = y_ref[...] + lax.dot_general(
        p0_ref[...] + p1_ref[...], wr_ref[...],
        (((1,), (1,)), ((), ())), preferred_element_type=jnp.float32)


def kernel(x, edge_index, edge_weight, cat_list, Wl, bl, Wr, br, edge_emb):
    del cat_list
    x = x.astype(jnp.float32)
    pad = EPAD - E
    src = edge_index[0].astype(jnp.int32)
    w = edge_weight.astype(jnp.int32)
    dst = edge_index[1].astype(jnp.int32)
    pad_idx = jnp.arange(pad, dtype=jnp.int32)
    pad_packed = (pad_idx % N) | ((N + pad_idx % (A - N)) << 17)
    packed = jnp.concatenate(
        [(w * N + src) | (dst << 17), pad_packed]).reshape(NROW, C)

    bias = (bl + br).astype(jnp.float32)[None, :]
    BLK = 2000
    nblk = N // BLK
    xs, y = pl.pallas_call(
        _tc_scale_kernel,
        grid=(nblk, W),
        in_specs=[
            pl.BlockSpec((BLK, D), lambda i, v: (i, 0)),
            pl.BlockSpec((W, D), lambda i, v: (0, 0)),
            pl.BlockSpec((D, D), lambda i, v: (0, 0)),
            pl.BlockSpec((1, D), lambda i, v: (0, 0)),
        ],
        out_specs=[
            pl.BlockSpec((BLK, D), lambda i, v: (v * nblk + i, 0)),
            pl.BlockSpec((BLK, D), lambda i, v: (i, 0)),
        ],
        out_shape=[
            jax.ShapeDtypeStruct((W * N, D), jnp.float32),
            jax.ShapeDtypeStruct((N, D), jnp.float32),
        ],
    )(x, edge_emb, Wl, bias)

    sc = functools.partial(
        pl.kernel,
        out_type=jax.ShapeDtypeStruct((2, A, D), jnp.float32),
        mesh=plsc.VectorSubcoreMesh(core_axis_name="c", subcore_axis_name="s"),
        scratch_types=[
            pltpu.VMEM((GRP, C), jnp.int32),
            pltpu.VMEM((GRP, C), jnp.int32),
            pltpu.VMEM((GRP, C), jnp.int32),
            pltpu.VMEM((C, D), jnp.float32),
            pltpu.VMEM((C, D), jnp.float32),
            pltpu.VMEM((C, D), jnp.float32),
            pltpu.VMEM((C, D), jnp.float32),
            pltpu.VMEM_SHARED((A, D), jnp.float32),
            pltpu.SemaphoreType.DMA,
            pltpu.SemaphoreType.DMA,
            pltpu.SemaphoreType.DMA,
            pltpu.SemaphoreType.DMA,
            pltpu.SemaphoreType.DMA,
            pltpu.SemaphoreType.DMA,
            pltpu.SemaphoreType.DMA,
            pltpu.SemaphoreType.DMA,
            pltpu.SemaphoreType.DMA,
        ],
    )(_sc_prop_kernel)
    prop2 = sc(packed, xs)

    out = pl.pallas_call(
        _tc_out_kernel,
        grid=(nblk,),
        in_specs=[
            pl.BlockSpec((BLK, D), lambda i: (i, 0)),
            pl.BlockSpec((None, BLK, D), lambda i: (0, i, 0)),
            pl.BlockSpec((None, BLK, D), lambda i: (1, i, 0)),
            pl.BlockSpec((D, D), lambda i: (0, 0)),
        ],
        out_specs=pl.BlockSpec((BLK, D), lambda i: (i, 0)),
        out_shape=jax.ShapeDtypeStruct((N, D), jnp.float32),
    )(y, prop2, prop2, Wr)
    return out
